# Initial kernel scaffold; baseline (speedup 1.0000x reference)
import jax
import jax.numpy as jnp
from jax.experimental import pallas as pl


def kernel(point_cloud, query_cloud, temperature):
    raise NotImplementedError("write your pallas kernel here")


# speedup vs baseline: 26.1148x; 26.1148x over previous
import jax
import jax.numpy as jnp
from jax.experimental import pallas as pl
from jax.experimental.pallas import tpu as pltpu

GROUP_SIZE = 16
MIN_SIGMA = 1e-4
MBLK = 128


def _body(p_ref, q_ref, inv_sigma_ref, out_ref):
    p = p_ref[0]
    q = q_ref[0]
    px = p[0:1, :]
    py = p[1:2, :]
    pz = p[2:3, :]
    qx = q[:, 0:1]
    qy = q[:, 1:2]
    qz = q[:, 2:3]
    p2 = px * px + py * py + pz * pz
    q2 = qx * qx + qy * qy + qz * qz
    bf = jnp.bfloat16
    f32 = jnp.float32
    pxb = px.astype(bf).astype(f32)
    pyb = py.astype(bf).astype(f32)
    pzb = pz.astype(bf).astype(f32)
    qxb = qx.astype(bf).astype(f32)
    qyb = qy.astype(bf).astype(f32)
    qzb = qz.astype(bf).astype(f32)
    csel = qxb * pxb + qyb * pyb + qzb * pzb
    dsel = q2 - 2.0 * csel + p2
    c = qx * px + qy * py + qz * pz
    d2 = q2 - 2.0 * c + p2

    inf = jnp.float32(jnp.inf)
    t = jnp.min(dsel, axis=1, keepdims=True)
    for _ in range(GROUP_SIZE - 1):
        t = jnp.min(jnp.where(dsel > t, dsel, inf), axis=1, keepdims=True)

    mask = dsel <= t
    dmin = jnp.min(jnp.where(mask, d2, inf), axis=1, keepdims=True)
    inv_sigma = inv_sigma_ref[0]
    w = jnp.where(mask, jnp.exp((dmin - d2) * inv_sigma), 0.0)
    den = jnp.sum(w, axis=1, keepdims=True)
    nx = jnp.sum(w * px, axis=1, keepdims=True)
    ny = jnp.sum(w * py, axis=1, keepdims=True)
    nz = jnp.sum(w * pz, axis=1, keepdims=True)
    r = 1.0 / den
    out_ref[0] = jnp.concatenate([nx * r, ny * r, nz * r], axis=1)


def _build(B, N, M, interpret=False):
    grid = (B, M // MBLK)
    return pl.pallas_call(
        _body,
        grid=grid,
        in_specs=[
            pl.BlockSpec((1, 3, N), lambda b, j: (b, 0, 0)),
            pl.BlockSpec((1, MBLK, 3), lambda b, j: (b, j, 0)),
            pl.BlockSpec(memory_space=pltpu.SMEM),
        ],
        out_specs=pl.BlockSpec((1, MBLK, 3), lambda b, j: (b, j, 0)),
        out_shape=jax.ShapeDtypeStruct((B, M, 3), jnp.float32),
        interpret=interpret,
    )


def kernel(point_cloud, query_cloud, temperature):
    B, _, N = point_cloud.shape
    M = query_cloud.shape[2]
    qt = jnp.transpose(query_cloud, (0, 2, 1))
    sigma = jnp.maximum(temperature ** 2, jnp.asarray(MIN_SIGMA, jnp.float32))
    inv_sigma = (1.0 / sigma).reshape(1).astype(jnp.float32)
    out = _build(B, N, M)(point_cloud, qt, inv_sigma)
    return jnp.transpose(out, (0, 2, 1))
